# baseline (speedup 1.0000x reference)
import jax
import jax.numpy as jnp
from jax.experimental import pallas as pl
from jax.experimental.pallas import tpu as pltpu

_TAU = 0.02
_LOG2E = 1.4426950408889634
_LN2 = 0.6931471805599453
_SCALE = _LOG2E / _TAU
_EPS = 1e-12

_B = 1024


def _loss_kernel(ts_ref, sq_ref, sqc_ref, pm_ref, out_ref, sqn_ref, acc_ref):
    i = pl.program_id(0)
    ng = pl.num_programs(0)

    @pl.when(i == 0)
    def _():
        q = sq_ref[...]
        qs = jnp.sum(q * q, axis=1, keepdims=True)
        qinv = 1.0 / jnp.maximum(jnp.sqrt(qs), _EPS)
        sqn_ref[...] = (q * qinv).astype(jnp.bfloat16)
        acc_ref[0] = 0.0
        acc_ref[1] = jnp.sum(pm_ref[...]).astype(jnp.float32)

    cnt = acc_ref[1]
    base = (i * _B).astype(jnp.float32)

    @pl.when(base < cnt)
    def _():
        t = ts_ref[...]
        ssq = jnp.sum(t * t, axis=1, keepdims=True)
        inv = _SCALE / jnp.maximum(jnp.sqrt(ssq), _EPS)
        tb = (t * inv).astype(jnp.bfloat16)

        x = jax.lax.dot_general(
            tb, sqn_ref[...],
            dimension_numbers=(((1,), (1,)), ((), ())),
            preferred_element_type=jnp.float32,
        )

        s = jnp.sum(jnp.exp2(x), axis=1, keepdims=True)
        lse = jnp.log(s)

        qd = sqc_ref[...]
        qss = jnp.sum(qd * qd, axis=1, keepdims=True)
        qdn = (qd * (1.0 / jnp.maximum(jnp.sqrt(qss), _EPS))).astype(jnp.bfloat16)
        diag = jnp.sum(tb.astype(jnp.float32) * qdn.astype(jnp.float32),
                       axis=1, keepdims=True) * _LN2

        rid = jax.lax.broadcasted_iota(jnp.int32, (1, _B), 1)
        valid = (rid.astype(jnp.float32) < (cnt - base)).astype(jnp.float32)
        lp = jax.lax.dot_general(
            valid, diag - lse,
            dimension_numbers=(((1,), (0,)), ((), ())),
            preferred_element_type=jnp.float32,
            precision=jax.lax.Precision.HIGHEST,
        )
        acc_ref[0] += lp[0, 0]

    @pl.when(i == ng - 1)
    def _():
        out_ref[0, 0] = -acc_ref[0] / (acc_ref[1] + 1e-6)


def kernel(ts_out, seq_out, omega, patch_mask):
    del omega
    n, d = ts_out.shape
    g = n // _B

    idx = jnp.nonzero(patch_mask, size=n, fill_value=0)[0]
    ts_c = jnp.take(ts_out, idx, axis=0)
    sq_c = jnp.take(seq_out, idx, axis=0)
    pm_row = patch_mask.reshape(1, n)

    loss = pl.pallas_call(
        _loss_kernel,
        grid=(g,),
        in_specs=[
            pl.BlockSpec((_B, d), lambda i: (i, 0)),
            pl.BlockSpec((n, d), lambda i: (0, 0)),
            pl.BlockSpec((_B, d), lambda i: (i, 0)),
            pl.BlockSpec((1, n), lambda i: (0, 0)),
        ],
        out_specs=pl.BlockSpec((1, 1), lambda i: (0, 0),
                               memory_space=pltpu.SMEM),
        out_shape=jax.ShapeDtypeStruct((1, 1), jnp.float32),
        scratch_shapes=[
            pltpu.VMEM((n, d), jnp.bfloat16),
            pltpu.SMEM((2,), jnp.float32),
        ],
        compiler_params=pltpu.CompilerParams(
            dimension_semantics=("arbitrary",),
            vmem_limit_bytes=120 * 1024 * 1024,
        ),
        name="nce_loss",
    )(ts_c, seq_out, sq_c, pm_row)

    return loss[0, 0]


# speedup vs baseline: 8.7936x; 8.7936x over previous
import jax
import jax.numpy as jnp
from jax.experimental import pallas as pl
from jax.experimental.pallas import tpu as pltpu

_TAU = 0.02
_LOG2E = 1.4426950408889634
_LN2 = 0.6931471805599453
_SCALE = _LOG2E / _TAU
_EPS = 1e-12

_B = 1024


def _loss_kernel(ts_ref, sq_ref, pm_ref, out_ref, sqn_ref, acc_ref):
    i = pl.program_id(0)
    ng = pl.num_programs(0)

    @pl.when(i == 0)
    def _():
        q = sq_ref[...]
        qs = jnp.sum(q * q, axis=1, keepdims=True)
        qinv = jax.lax.rsqrt(jnp.maximum(qs, _EPS * _EPS))
        sqn_ref[...] = (q * qinv).astype(jnp.bfloat16)
        acc_ref[0] = 0.0
        acc_ref[1] = 0.0

    t = ts_ref[...]
    ssq = jnp.sum(t * t, axis=1, keepdims=True)
    inv = _SCALE * jax.lax.rsqrt(jnp.maximum(ssq, _EPS * _EPS))
    tb = (t * inv).astype(jnp.bfloat16)

    x = jax.lax.dot_general(
        tb, sqn_ref[...],
        dimension_numbers=(((1,), (1,)), ((), ())),
        preferred_element_type=jnp.float32,
    )

    s = jnp.sum(jnp.exp2(x), axis=1, keepdims=True)
    lse = jnp.log(s)

    sqd = sqn_ref[pl.ds(i * _B, _B), :].astype(jnp.float32)
    diag = jnp.sum(tb.astype(jnp.float32) * sqd, axis=1, keepdims=True) * _LN2

    pm = pm_ref[0].astype(jnp.float32)
    lp = jax.lax.dot_general(
        pm, diag - lse,
        dimension_numbers=(((1,), (0,)), ((), ())),
        preferred_element_type=jnp.float32,
    )

    acc_ref[0] += lp[0, 0]
    acc_ref[1] += jnp.sum(pm)

    @pl.when(i == ng - 1)
    def _():
        out_ref[0, 0] = -acc_ref[0] / (acc_ref[1] + 1e-6)


def kernel(ts_out, seq_out, omega, patch_mask):
    del omega
    n, d = ts_out.shape
    g = n // _B
    pm3 = patch_mask.reshape(g, 1, _B)

    loss = pl.pallas_call(
        _loss_kernel,
        grid=(g,),
        in_specs=[
            pl.BlockSpec((_B, d), lambda i: (i, 0)),
            pl.BlockSpec((n, d), lambda i: (0, 0)),
            pl.BlockSpec((1, 1, _B), lambda i: (i, 0, 0)),
        ],
        out_specs=pl.BlockSpec((1, 1), lambda i: (0, 0),
                               memory_space=pltpu.SMEM),
        out_shape=jax.ShapeDtypeStruct((1, 1), jnp.float32),
        scratch_shapes=[
            pltpu.VMEM((n, d), jnp.bfloat16),
            pltpu.SMEM((2,), jnp.float32),
        ],
        compiler_params=pltpu.CompilerParams(
            dimension_semantics=("arbitrary",),
            vmem_limit_bytes=120 * 1024 * 1024,
        ),
        name="nce_loss",
    )(ts_out, seq_out, pm3)

    return loss[0, 0]


# speedup vs baseline: 9.0768x; 1.0322x over previous
import jax
import jax.numpy as jnp
from jax.experimental import pallas as pl
from jax.experimental.pallas import tpu as pltpu

_TAU = 0.02
_LOG2E = 1.4426950408889634
_LN2 = 0.6931471805599453
_SCALE = _LOG2E / _TAU
_EPS = 1e-12

_B = 2048


def _loss_kernel(ts_ref, sq_ref, pm_ref, out_ref, sqn_ref, acc_ref):
    i = pl.program_id(0)
    ng = pl.num_programs(0)

    @pl.when(i == 0)
    def _():
        q = sq_ref[...]
        qs = jnp.sum(q * q, axis=1, keepdims=True)
        qinv = jax.lax.rsqrt(jnp.maximum(qs, _EPS * _EPS))
        sqn_ref[...] = (q * qinv).astype(jnp.bfloat16)
        acc_ref[0] = 0.0
        acc_ref[1] = 0.0

    t = ts_ref[...]
    ssq = jnp.sum(t * t, axis=1, keepdims=True)
    inv = _SCALE * jax.lax.rsqrt(jnp.maximum(ssq, _EPS * _EPS))
    tb = (t * inv).astype(jnp.bfloat16)

    x = jax.lax.dot_general(
        tb, sqn_ref[...],
        dimension_numbers=(((1,), (1,)), ((), ())),
        preferred_element_type=jnp.float32,
    )

    s = jnp.sum(jnp.exp2(x), axis=1, keepdims=True)
    lse = jnp.log(s)

    sqd = sqn_ref[pl.ds(i * _B, _B), :].astype(jnp.float32)
    diag = jnp.sum(tb.astype(jnp.float32) * sqd, axis=1, keepdims=True) * _LN2

    pm = pm_ref[0].astype(jnp.float32)
    lp = jax.lax.dot_general(
        pm, diag - lse,
        dimension_numbers=(((1,), (0,)), ((), ())),
        preferred_element_type=jnp.float32,
    )

    acc_ref[0] += lp[0, 0]
    acc_ref[1] += jnp.sum(pm)

    @pl.when(i == ng - 1)
    def _():
        out_ref[0, 0] = -acc_ref[0] / (acc_ref[1] + 1e-6)


def kernel(ts_out, seq_out, omega, patch_mask):
    del omega
    n, d = ts_out.shape
    g = n // _B
    pm3 = patch_mask.reshape(g, 1, _B)

    loss = pl.pallas_call(
        _loss_kernel,
        grid=(g,),
        in_specs=[
            pl.BlockSpec((_B, d), lambda i: (i, 0)),
            pl.BlockSpec((n, d), lambda i: (0, 0)),
            pl.BlockSpec((1, 1, _B), lambda i: (i, 0, 0)),
        ],
        out_specs=pl.BlockSpec((1, 1), lambda i: (0, 0),
                               memory_space=pltpu.SMEM),
        out_shape=jax.ShapeDtypeStruct((1, 1), jnp.float32),
        scratch_shapes=[
            pltpu.VMEM((n, d), jnp.bfloat16),
            pltpu.SMEM((2,), jnp.float32),
        ],
        compiler_params=pltpu.CompilerParams(
            dimension_semantics=("arbitrary",),
            vmem_limit_bytes=120 * 1024 * 1024,
        ),
        name="nce_loss",
    )(ts_out, seq_out, pm3)

    return loss[0, 0]
